# baseline (speedup 1.0000x reference)
import functools

import jax
import jax.numpy as jnp
from jax import lax
from jax.experimental import pallas as pl
from jax.experimental.pallas import tpu as pltpu
from jax.experimental.pallas import tpu_sc as plsc

N_USERS = 4000
N_ITEMS = 6000
N = N_USERS + N_ITEMS
R = 3
D = 128
L = 3
E = 320000

NC = 2
NS = 16
DH = D // NC
CHUNK = 128
RING = 4
NBLK = 2
NCHUNK = 80
NGRP = NCHUNK // RING
EPT = NBLK * NCHUNK * CHUNK
EPAD = NS * EPT
N_PAD = 10240
ROWS_PT = N_PAD // NS

BN = 400
NB = N // BN


def _leaky(x):
    return jnp.where(x >= 0, x, 0.01 * x)


def _dot(a, b):
    return lax.dot_general(a, b, (((1,), (0,)), ((), ())),
                           preferred_element_type=jnp.float32)


def _tc_rela(rel_emb, W_rel):
    def body(re_ref, w_ref, rt_ref, rm_ref):
        r = re_ref[...]
        acc = r
        rt_ref[0] = r
        for k in range(L):
            r = _dot(r, w_ref[k])
            rt_ref[k + 1] = r
            acc = acc + r
        rm_ref[...] = (acc * (1.0 / (L + 1)))[:, None, :]

    return pl.pallas_call(
        body,
        grid=(1,),
        in_specs=[pl.BlockSpec((R, D), lambda b: (0, 0)),
                  pl.BlockSpec((L, D, D), lambda b: (0, 0, 0))],
        out_specs=[pl.BlockSpec((L + 1, R, D), lambda b: (0, 0, 0)),
                   pl.BlockSpec((R, 1, D), lambda b: (0, 0, 0))],
        out_shape=[jax.ShapeDtypeStruct((L + 1, R, D), jnp.float32),
                   jax.ShapeDtypeStruct((R, 1, D), jnp.float32)],
    )(rel_emb, W_rel)


def _split_store(y_ref, i, y):
    for cc in range(NC):
        y_ref[cc, i] = y[:, cc * DH:(cc + 1) * DH]


def _tc_pre(base, rela, W):
    def body(b_ref, r_ref, w_ref, y_ref):
        x = b_ref[...]
        w = w_ref[...]
        for i in range(R):
            _split_store(y_ref, i, _dot(x * r_ref[i][None, :], w))

    return pl.pallas_call(
        body,
        grid=(NB,),
        in_specs=[pl.BlockSpec((BN, D), lambda b: (b, 0)),
                  pl.BlockSpec((R, D), lambda b: (0, 0)),
                  pl.BlockSpec((D, D), lambda b: (0, 0))],
        out_specs=pl.BlockSpec((NC, R, BN, DH), lambda b: (0, 0, b, 0)),
        out_shape=jax.ShapeDtypeStruct((NC, R, N, DH), jnp.float32),
    )(base, rela, W)


def _tc_mid(part, acc, rela, W):
    def body(p_ref, a_ref, r_ref, w_ref, y_ref, ao_ref):
        w = w_ref[...]
        for i in range(R):
            e = _leaky(jnp.concatenate([p_ref[0, i], p_ref[1, i]], axis=-1))
            ao_ref[i] = a_ref[i] + e
            _split_store(y_ref, i, _dot(e * r_ref[i][None, :], w))

    return pl.pallas_call(
        body,
        grid=(NB,),
        in_specs=[pl.BlockSpec((NC, R, BN, DH), lambda b: (0, 0, b, 0)),
                  pl.BlockSpec((R, BN, D), lambda b: (0, b, 0)),
                  pl.BlockSpec((R, D), lambda b: (0, 0)),
                  pl.BlockSpec((D, D), lambda b: (0, 0))],
        out_specs=[pl.BlockSpec((NC, R, BN, DH), lambda b: (0, 0, b, 0)),
                   pl.BlockSpec((R, BN, D), lambda b: (0, b, 0))],
        out_shape=[jax.ShapeDtypeStruct((NC, R, N, DH), jnp.float32),
                   jax.ShapeDtypeStruct((R, N, D), jnp.float32)],
    )(part, acc, rela, W)


def _tc_fin(part, acc, base):
    def body(p_ref, a_ref, b_ref, o_ref):
        x = b_ref[...]
        for i in range(R):
            e = _leaky(jnp.concatenate([p_ref[0, i], p_ref[1, i]], axis=-1))
            o_ref[:, i, :] = (x + a_ref[i] + e) * 0.25

    return pl.pallas_call(
        body,
        grid=(NB,),
        in_specs=[pl.BlockSpec((NC, R, BN, DH), lambda b: (0, 0, b, 0)),
                  pl.BlockSpec((R, BN, D), lambda b: (0, b, 0)),
                  pl.BlockSpec((BN, D), lambda b: (b, 0))],
        out_specs=pl.BlockSpec((BN, R, D), lambda b: (b, 0, 0)),
        out_shape=jax.ShapeDtypeStruct((N, R, D), jnp.float32),
    )(part, acc, base)


def _sc_spmm(yall, cols, rows, vals, zeros):
    mesh = plsc.VectorSubcoreMesh(core_axis_name="c", subcore_axis_name="s",
                                  num_cores=NC, num_subcores=NS)

    @functools.partial(
        pl.kernel,
        out_type=jax.ShapeDtypeStruct((NC, R, N_PAD, DH), jnp.float32),
        mesh=mesh,
        scratch_types=[
            pltpu.VMEM_SHARED((N_PAD, DH), jnp.float32),
            pltpu.VMEM((NCHUNK, CHUNK), jnp.int32),
            pltpu.VMEM((NCHUNK, CHUNK), jnp.int32),
            pltpu.VMEM((NCHUNK, CHUNK), jnp.float32),
            pltpu.VMEM((RING, CHUNK, DH), jnp.float32),
        ] + [pltpu.SemaphoreType.DMA] * (2 * RING),
        compiler_params=pltpu.CompilerParams(use_tc_tiling_on_sc=False),
    )
    def k(yr, colsr, rowsr, valsr, zerosr, partr,
          accum, colbuf, rowbuf, valbuf, gbuf, *sems):
        gsem = sems[:RING]
        ssem = sems[RING:]
        c = lax.axis_index("c")
        s = lax.axis_index("s")
        rslice = pl.ds(pl.multiple_of(s * ROWS_PT, 8), ROWS_PT)

        def issue_gather(jj, r):
            pltpu.async_copy(yr.at[colbuf.at[jj]], gbuf.at[r], gsem[r])

        def wait_gather(jj, r):
            pltpu.make_async_copy(
                yr.at[colbuf.at[jj]], gbuf.at[r], gsem[r]).wait()

        def issue_scatter(jj, r):
            pltpu.async_copy(gbuf.at[r], accum.at[rowbuf.at[jj]],
                             ssem[r], add=True)

        def wait_scatter(jj, r):
            pltpu.make_async_copy(
                gbuf.at[r], accum.at[rowbuf.at[jj]], ssem[r]).wait()

        for i in range(R):
            pltpu.sync_copy(zerosr.at[rslice], accum.at[rslice])
            plsc.subcore_barrier()
            for sb in range(NBLK):
                pltpu.sync_copy(colsr.at[c, i, s, sb], colbuf)
                pltpu.sync_copy(rowsr.at[i, s, sb], rowbuf)
                pltpu.sync_copy(valsr.at[i, s, sb], valbuf)

                issue_gather(0, 0)
                issue_gather(1, 1)

                def grp_body(gg, _):
                    for h in range(RING):
                        g = RING * gg + h
                        rg = (h + 2) % RING
                        if h < 2:
                            @pl.when(gg >= 1)
                            def _(g=g, rg=rg):
                                wait_scatter(g - 2, rg)
                                issue_gather(g + 2, rg)

                            @pl.when(gg == 0)
                            def _(g=g, rg=rg):
                                issue_gather(g + 2, rg)
                        else:
                            wait_scatter(g - 2, rg)

                            @pl.when(gg < NGRP - 1)
                            def _(g=g, rg=rg):
                                issue_gather(g + 2, rg)

                        wait_gather(g, h)

                        def scale_body(t, _, h=h, g=g):
                            vv = valbuf[g, pl.ds(t * 16, 16)]
                            for u in range(16):
                                v = vv[u]
                                e = t * 16 + u
                                for q in range(DH // 16):
                                    sl = pl.ds(q * 16, 16)
                                    gbuf[h, e, sl] = gbuf[h, e, sl] * v
                            return 0

                        lax.fori_loop(0, CHUNK // 16, scale_body, 0)
                        issue_scatter(g, h)
                    return 0

                lax.fori_loop(0, NGRP, grp_body, 0)
                wait_scatter(NCHUNK - 2, 2)
                wait_scatter(NCHUNK - 1, 3)
            plsc.subcore_barrier()
            pltpu.sync_copy(accum.at[rslice], partr.at[c, i, rslice])

    return k(yall, cols, rows, vals, zeros)


def kernel(adj_idx, adj_val, user_embedding, item_embedding,
           relation_embedding, W_gc, W_rel):
    base = jnp.concatenate([user_embedding, item_embedding], axis=0)
    cols = adj_idx[:, 1, :].astype(jnp.int32)
    rows = adj_idx[:, 0, :].astype(jnp.int32)
    pad = EPAD - E
    colsf = cols + (jnp.arange(R, dtype=jnp.int32) * N)[:, None]
    colsf = jnp.pad(colsf, ((0, 0), (0, pad)))
    colsf = colsf.reshape(R, NS, NBLK, NCHUNK, CHUNK)
    cols_p = jnp.stack([colsf, colsf + R * N], axis=0)
    rows_p = jnp.pad(rows, ((0, 0), (0, pad))).reshape(
        R, NS, NBLK, NCHUNK, CHUNK)
    vals_p = jnp.pad(adj_val, ((0, 0), (0, pad))).reshape(
        R, NS, NBLK, NCHUNK, CHUNK)
    zeros = jnp.zeros((N_PAD, DH), jnp.float32)

    rt, rmean = _tc_rela(relation_embedding, W_rel)

    yall = _tc_pre(base, rt[0], W_gc[0])
    acc = jnp.zeros((R, N, D), jnp.float32)
    for k in range(1, L):
        part = _sc_spmm(yall.reshape(NC * R * N, DH),
                        cols_p, rows_p, vals_p, zeros)
        yall, acc = _tc_mid(part, acc, rt[k], W_gc[k])
    part = _sc_spmm(yall.reshape(NC * R * N, DH),
                    cols_p, rows_p, vals_p, zeros)
    all_emb = _tc_fin(part, acc, base)

    u_g = all_emb[:N_USERS]
    i_g = jnp.concatenate(
        [all_emb[N_USERS:], jnp.zeros((1, R, D), jnp.float32)], axis=0)
    return (u_g, i_g, rmean)
